# baseline (speedup 1.0000x reference)
import jax
import jax.numpy as jnp
from jax.experimental import pallas as pl
from jax.experimental.pallas import tpu as pltpu

N_EDGES = 8192
N_NODES = 8192
N_FACES = 2048
HID = 32
ROW_BLK = 256
NB = N_EDGES // ROW_BLK
I2_BLK = 512
NI2 = N_EDGES // I2_BLK
C_ST0 = 0
C_ST1 = 32
C_Y1 = 64
C_X1L0 = 96


def _elu(x):
    return jnp.where(x > 0, x, jnp.exp(x) - 1.0)


def _dot(a, b):
    return jnp.dot(a, b, preferred_element_type=jnp.float32)


def _proj_body(x0_ref, x1_ref, x2_ref, w0_ref, b0_ref, w1_ref, b1_ref,
               w2_ref, b2_ref, x0p_ref, x1p_ref, x2p_ref):
    x0p_ref[...] = _elu(_dot(x0_ref[...], w0_ref[...]) + b0_ref[...])
    x1p_ref[...] = _elu(_dot(x1_ref[...], w1_ref[...]) + b1_ref[...])
    x2p_ref[...] = _elu(_dot(x2_ref[...], w2_ref[...]) + b2_ref[...])


def _body(x0p_ref, x1p_ref, x2p_ref, i1t_ref, i2_ref, a_ref,
          w11a_ref, w21a_ref, w01a_ref, uwa_ref, uba_ref,
          w11b_ref, w21b_ref, w01b_ref, uwb_ref, ubb_ref,
          x1_out, s_ref):
    i = pl.program_id(0)

    @pl.when(i < NI2)
    def _():
        row = i * I2_BLK
        s_ref[pl.ds(row, I2_BLK), C_Y1:C_Y1 + HID] = _dot(
            i2_ref[...], x2p_ref[...])

    @pl.when((i >= NI2) & (i < NI2 + NB))
    def _():
        row = (i - NI2) * ROW_BLK
        p0 = _dot(i1t_ref[...], x0p_ref[...])
        p2 = s_ref[pl.ds(row, ROW_BLK), C_Y1:C_Y1 + HID]
        s_ref[pl.ds(row, ROW_BLK), C_ST0:C_ST0 + HID] = (
            _elu(_dot(p0, w01a_ref[...])) + _elu(_dot(p2, w21a_ref[...])))
        s_ref[pl.ds(row, ROW_BLK), C_ST1:C_ST1 + HID] = (
            _elu(_dot(p0, w01b_ref[...])) + _elu(_dot(p2, w21b_ref[...])))

    @pl.when(i == NI2 + NB)
    def _():
        s_ref[:, C_Y1:C_Y1 + HID] = _dot(x1p_ref[...], w11a_ref[...])

    @pl.when((i >= NI2 + NB) & (i < NI2 + 2 * NB))
    def _():
        row = (i - NI2 - NB) * ROW_BLK
        x_up = _elu(_dot(a_ref[...], s_ref[:, C_Y1:C_Y1 + HID]))
        agg = x_up + s_ref[pl.ds(row, ROW_BLK), C_ST0:C_ST0 + HID]
        s_ref[pl.ds(row, ROW_BLK), C_X1L0:C_X1L0 + HID] = _elu(
            _dot(agg, uwa_ref[...]) + uba_ref[...])

    @pl.when(i == NI2 + 2 * NB)
    def _():
        s_ref[:, C_Y1:C_Y1 + HID] = _dot(
            s_ref[:, C_X1L0:C_X1L0 + HID], w11b_ref[...])

    @pl.when(i >= NI2 + 2 * NB)
    def _():
        row = (i - NI2 - 2 * NB) * ROW_BLK
        x_up = _elu(_dot(a_ref[...], s_ref[:, C_Y1:C_Y1 + HID]))
        agg = x_up + s_ref[pl.ds(row, ROW_BLK), C_ST1:C_ST1 + HID]
        x1_out[...] = _elu(_dot(agg, uwb_ref[...]) + ubb_ref[...])


@jax.jit
def kernel(x_0, x_1, x_2, adjacency_0, incidence_2, incidence_1_t,
           proj0_w, proj0_b, proj1_w, proj1_b, proj2_w, proj2_b,
           l0_w11, l0_w21, l0_w01, l0_uw, l0_ub,
           l1_w11, l1_w21, l1_w01, l1_uw, l1_ub):
    f32 = jnp.float32
    const2 = lambda i: (0, 0)

    x0p, x1p, x2p = pl.pallas_call(
        _proj_body,
        out_shape=(
            jax.ShapeDtypeStruct((N_NODES, HID), f32),
            jax.ShapeDtypeStruct((N_EDGES, HID), f32),
            jax.ShapeDtypeStruct((N_FACES, HID), f32),
        ),
    )(x_0, x_1, x_2, proj0_w, proj0_b.reshape(1, HID),
      proj1_w, proj1_b.reshape(1, HID), proj2_w, proj2_b.reshape(1, HID))

    def i2_map(i):
        return (jnp.minimum(i, NI2 - 1), 0)

    def i1t_map(i):
        return (jnp.clip(i - NI2, 0, NB - 1), 0)

    def a_map(i):
        return (jnp.where(i < NI2 + NB, 0,
                          jnp.where(i < NI2 + 2 * NB, i - NI2 - NB,
                                    i - NI2 - 2 * NB)), 0)

    def out_map(i):
        return (jnp.maximum(i - NI2 - 2 * NB, 0), 0)

    small = [pl.BlockSpec((HID, HID), const2)] * 4 + [
        pl.BlockSpec((1, HID), const2)]

    x1_final = pl.pallas_call(
        _body,
        grid=(NI2 + 3 * NB,),
        in_specs=[
            pl.BlockSpec((N_NODES, HID), const2),
            pl.BlockSpec((N_EDGES, HID), const2),
            pl.BlockSpec((N_FACES, HID), const2),
            pl.BlockSpec((ROW_BLK, N_NODES), i1t_map),
            pl.BlockSpec((I2_BLK, N_FACES), i2_map),
            pl.BlockSpec((ROW_BLK, N_EDGES), a_map),
        ] + small + small,
        out_specs=pl.BlockSpec((ROW_BLK, HID), out_map),
        out_shape=jax.ShapeDtypeStruct((N_EDGES, HID), f32),
        scratch_shapes=[
            pltpu.VMEM((N_EDGES, 128), f32),
        ],
        compiler_params=pltpu.CompilerParams(
            dimension_semantics=("arbitrary",),
            vmem_limit_bytes=63 * 1024 * 1024),
    )(x0p, x1p, x2p, incidence_1_t, incidence_2, adjacency_0,
      l0_w11, l0_w21, l0_w01, l0_uw, l0_ub.reshape(1, HID),
      l1_w11, l1_w21, l1_w01, l1_uw, l1_ub.reshape(1, HID))

    return (x0p, x1_final, x2p)


# speedup vs baseline: 1.0389x; 1.0389x over previous
import jax
import jax.numpy as jnp
from jax.experimental import pallas as pl
from jax.experimental.pallas import tpu as pltpu

N_EDGES = 8192
N_NODES = 8192
N_FACES = 2048
HID = 32
ROW_BLK = 256
NB = N_EDGES // ROW_BLK
I2_BLK = N_FACES // NB * (N_EDGES // N_FACES)
assert I2_BLK * NB == N_EDGES


def _elu(x):
    return jnp.where(x > 0, x, jnp.exp(x) - 1.0)


def _dot(a, b):
    return jnp.dot(a, b, preferred_element_type=jnp.float32)


def _proj_body(x0_ref, x1_ref, x2_ref, w0_ref, b0_ref, w1_ref, b1_ref,
               w2_ref, b2_ref, x0p_ref, x1p_ref, x2p_ref):
    x0p_ref[...] = _elu(_dot(x0_ref[...], w0_ref[...]) + b0_ref[...])
    x1p_ref[...] = _elu(_dot(x1_ref[...], w1_ref[...]) + b1_ref[...])
    x2p_ref[...] = _elu(_dot(x2_ref[...], w2_ref[...]) + b2_ref[...])


def _body(x0p_ref, x1p_ref, x2p_ref, i1t_ref, i2_ref, a_ref,
          w11a_ref, w01ab_ref, w21ab_ref, uwa_ref, uba_ref,
          w11b_ref, uwb_ref, ubb_ref,
          x1_out, y1a_ref, y1b_ref, st01_ref):
    i = pl.program_id(0)

    @pl.when(i < NB)
    def _():
        row = i * ROW_BLK
        p2 = _dot(i2_ref[...], x2p_ref[...])
        p0 = _dot(i1t_ref[...], x0p_ref[...])
        st01_ref[pl.ds(row, ROW_BLK), :] = (
            _elu(_dot(p0, w01ab_ref[...])) + _elu(_dot(p2, w21ab_ref[...])))
        y1a_ref[pl.ds(row, ROW_BLK), :] = _dot(
            x1p_ref[pl.ds(row, ROW_BLK), :], w11a_ref[...])

    @pl.when((i >= NB) & (i < 2 * NB))
    def _():
        row = (i - NB) * ROW_BLK
        x_up = _elu(_dot(a_ref[...], y1a_ref[...]))
        agg = x_up + st01_ref[pl.ds(row, ROW_BLK), 0:HID]
        x1l0 = _elu(_dot(agg, uwa_ref[...]) + uba_ref[...])
        y1b_ref[pl.ds(row, ROW_BLK), :] = _dot(x1l0, w11b_ref[...])

    @pl.when(i >= 2 * NB)
    def _():
        row = (i - 2 * NB) * ROW_BLK
        x_up = _elu(_dot(a_ref[...], y1b_ref[...]))
        agg = x_up + st01_ref[pl.ds(row, ROW_BLK), HID:2 * HID]
        x1_out[...] = _elu(_dot(agg, uwb_ref[...]) + ubb_ref[...])


@jax.jit
def kernel(x_0, x_1, x_2, adjacency_0, incidence_2, incidence_1_t,
           proj0_w, proj0_b, proj1_w, proj1_b, proj2_w, proj2_b,
           l0_w11, l0_w21, l0_w01, l0_uw, l0_ub,
           l1_w11, l1_w21, l1_w01, l1_uw, l1_ub):
    f32 = jnp.float32
    const2 = lambda i: (0, 0)

    x0p, x1p, x2p = pl.pallas_call(
        _proj_body,
        out_shape=(
            jax.ShapeDtypeStruct((N_NODES, HID), f32),
            jax.ShapeDtypeStruct((N_EDGES, HID), f32),
            jax.ShapeDtypeStruct((N_FACES, HID), f32),
        ),
    )(x_0, x_1, x_2, proj0_w, proj0_b.reshape(1, HID),
      proj1_w, proj1_b.reshape(1, HID), proj2_w, proj2_b.reshape(1, HID))

    w01ab = jnp.concatenate([l0_w01, l1_w01], axis=1)
    w21ab = jnp.concatenate([l0_w21, l1_w21], axis=1)

    def i2_map(i):
        return (jnp.minimum(i, NB - 1), 0)

    def i1t_map(i):
        return (jnp.minimum(i, NB - 1), 0)

    def a_map(i):
        return (jnp.where(i < NB, 0,
                          jnp.where(i < 2 * NB, i - NB, i - 2 * NB)), 0)

    def out_map(i):
        return (jnp.maximum(i - 2 * NB, 0), 0)

    x1_final = pl.pallas_call(
        _body,
        grid=(3 * NB,),
        in_specs=[
            pl.BlockSpec((N_NODES, HID), const2),
            pl.BlockSpec((N_EDGES, HID), const2),
            pl.BlockSpec((N_FACES, HID), const2),
            pl.BlockSpec((ROW_BLK, N_NODES), i1t_map),
            pl.BlockSpec((I2_BLK, N_FACES), i2_map),
            pl.BlockSpec((ROW_BLK, N_EDGES), a_map),
            pl.BlockSpec((HID, HID), const2),
            pl.BlockSpec((HID, 2 * HID), const2),
            pl.BlockSpec((HID, 2 * HID), const2),
            pl.BlockSpec((HID, HID), const2),
            pl.BlockSpec((1, HID), const2),
            pl.BlockSpec((HID, HID), const2),
            pl.BlockSpec((HID, HID), const2),
            pl.BlockSpec((1, HID), const2),
        ],
        out_specs=pl.BlockSpec((ROW_BLK, HID), out_map),
        out_shape=jax.ShapeDtypeStruct((N_EDGES, HID), f32),
        scratch_shapes=[
            pltpu.VMEM((N_EDGES, HID), f32),
            pltpu.VMEM((N_EDGES, HID), f32),
            pltpu.VMEM((N_EDGES, 2 * HID), f32),
        ],
        compiler_params=pltpu.CompilerParams(
            dimension_semantics=("arbitrary",),
            vmem_limit_bytes=63 * 1024 * 1024),
    )(x0p, x1p, x2p, incidence_1_t, incidence_2, adjacency_0,
      l0_w11, w01ab, w21ab, l0_uw, l0_ub.reshape(1, HID),
      l1_w11, l1_uw, l1_ub.reshape(1, HID))

    return (x0p, x1_final, x2p)


# speedup vs baseline: 1.0567x; 1.0171x over previous
import jax
import jax.numpy as jnp
from jax.experimental import pallas as pl
from jax.experimental.pallas import tpu as pltpu

N_EDGES = 8192
N_NODES = 8192
N_FACES = 2048
HID = 32
ROW_BLK = 256
NB = N_EDGES // ROW_BLK
I2_BLK = N_FACES // NB * (N_EDGES // N_FACES)
assert I2_BLK * NB == N_EDGES
C_ST0 = 0
C_ST1 = HID
C_X1P = 2 * HID


def _elu(x):
    return jnp.where(x > 0, x, jnp.exp(x) - 1.0)


def _dot(a, b):
    return jnp.dot(a, b, preferred_element_type=jnp.float32)


def _body(x0_ref, x1_ref, x2_ref, i1t_ref, i2_ref, a_ref,
          w0_ref, b0_ref, w1_ref, b1_ref, w2_ref, b2_ref,
          w11a_ref, w01ab_ref, w21ab_ref, uwa_ref, uba_ref,
          w11b_ref, uwb_ref, ubb_ref,
          x0p_out, x1_out, x2p_out, y1a_ref, y1b_ref, s_ref,
          x0p16_ref, x2p16_ref):
    i = pl.program_id(0)
    bf16 = jnp.bfloat16

    @pl.when(i == 0)
    def _():
        x0p = _elu(_dot(x0_ref[...], w0_ref[...]) + b0_ref[...])
        x0p_out[...] = x0p
        x0p16_ref[...] = x0p.astype(bf16)
        x2p = _elu(_dot(x2_ref[...], w2_ref[...]) + b2_ref[...])
        x2p_out[...] = x2p
        x2p16_ref[...] = x2p.astype(bf16)
        s_ref[:, C_X1P:C_X1P + HID] = _elu(
            _dot(x1_ref[...], w1_ref[...]) + b1_ref[...])

    @pl.when(i < NB)
    def _():
        row = i * ROW_BLK
        p2 = _dot(i2_ref[...].astype(bf16), x2p16_ref[...])
        p0 = _dot(i1t_ref[...].astype(bf16), x0p16_ref[...])
        s_ref[pl.ds(row, ROW_BLK), C_ST0:C_ST0 + 2 * HID] = (
            _elu(_dot(p0, w01ab_ref[...])) + _elu(_dot(p2, w21ab_ref[...])))
        y1a_ref[pl.ds(row, ROW_BLK), :] = _dot(
            s_ref[pl.ds(row, ROW_BLK), C_X1P:C_X1P + HID],
            w11a_ref[...]).astype(bf16)

    @pl.when((i >= NB) & (i < 2 * NB))
    def _():
        row = (i - NB) * ROW_BLK
        x_up = _elu(_dot(a_ref[...].astype(bf16), y1a_ref[...]))
        agg = x_up + s_ref[pl.ds(row, ROW_BLK), C_ST0:C_ST0 + HID]
        x1l0 = _elu(_dot(agg, uwa_ref[...]) + uba_ref[...])
        y1b_ref[pl.ds(row, ROW_BLK), :] = _dot(
            x1l0, w11b_ref[...]).astype(bf16)

    @pl.when(i >= 2 * NB)
    def _():
        row = (i - 2 * NB) * ROW_BLK
        x_up = _elu(_dot(a_ref[...].astype(bf16), y1b_ref[...]))
        agg = x_up + s_ref[pl.ds(row, ROW_BLK), C_ST1:C_ST1 + HID]
        x1_out[...] = _elu(_dot(agg, uwb_ref[...]) + ubb_ref[...])


@jax.jit
def kernel(x_0, x_1, x_2, adjacency_0, incidence_2, incidence_1_t,
           proj0_w, proj0_b, proj1_w, proj1_b, proj2_w, proj2_b,
           l0_w11, l0_w21, l0_w01, l0_uw, l0_ub,
           l1_w11, l1_w21, l1_w01, l1_uw, l1_ub):
    f32 = jnp.float32
    const2 = lambda i: (0, 0)
    C0 = x_0.shape[1]
    C1 = x_1.shape[1]
    C2 = x_2.shape[1]

    w01ab = jnp.concatenate([l0_w01, l1_w01], axis=1)
    w21ab = jnp.concatenate([l0_w21, l1_w21], axis=1)

    def i2_map(i):
        return (jnp.minimum(i, NB - 1), 0)

    def i1t_map(i):
        return (jnp.minimum(i, NB - 1), 0)

    def a_map(i):
        return (jnp.where(i < NB, 0,
                          jnp.where(i < 2 * NB, i - NB, i - 2 * NB)), 0)

    def out_map(i):
        return (jnp.maximum(i - 2 * NB, 0), 0)

    wsq = lambda: pl.BlockSpec((HID, HID), const2)
    bias = lambda: pl.BlockSpec((1, HID), const2)

    x0p, x1_final, x2p = pl.pallas_call(
        _body,
        grid=(3 * NB,),
        in_specs=[
            pl.BlockSpec((N_NODES, C0), const2),
            pl.BlockSpec((N_EDGES, C1), const2),
            pl.BlockSpec((N_FACES, C2), const2),
            pl.BlockSpec((ROW_BLK, N_NODES), i1t_map),
            pl.BlockSpec((I2_BLK, N_FACES), i2_map),
            pl.BlockSpec((ROW_BLK, N_EDGES), a_map),
            pl.BlockSpec((C0, HID), const2),
            bias(),
            pl.BlockSpec((C1, HID), const2),
            bias(),
            pl.BlockSpec((C2, HID), const2),
            bias(),
            wsq(),
            pl.BlockSpec((HID, 2 * HID), const2),
            pl.BlockSpec((HID, 2 * HID), const2),
            wsq(),
            bias(),
            wsq(),
            wsq(),
            bias(),
        ],
        out_specs=(
            pl.BlockSpec((N_NODES, HID), const2),
            pl.BlockSpec((ROW_BLK, HID), out_map),
            pl.BlockSpec((N_FACES, HID), const2),
        ),
        out_shape=(
            jax.ShapeDtypeStruct((N_NODES, HID), f32),
            jax.ShapeDtypeStruct((N_EDGES, HID), f32),
            jax.ShapeDtypeStruct((N_FACES, HID), f32),
        ),
        scratch_shapes=[
            pltpu.VMEM((N_EDGES, HID), jnp.bfloat16),
            pltpu.VMEM((N_EDGES, HID), jnp.bfloat16),
            pltpu.VMEM((N_EDGES, 3 * HID), f32),
            pltpu.VMEM((N_NODES, HID), jnp.bfloat16),
            pltpu.VMEM((N_FACES, HID), jnp.bfloat16),
        ],
        compiler_params=pltpu.CompilerParams(
            dimension_semantics=("arbitrary",),
            vmem_limit_bytes=63 * 1024 * 1024),
    )(x_0, x_1, x_2, incidence_1_t, incidence_2, adjacency_0,
      proj0_w, proj0_b.reshape(1, HID), proj1_w, proj1_b.reshape(1, HID),
      proj2_w, proj2_b.reshape(1, HID),
      l0_w11, w01ab, w21ab, l0_uw, l0_ub.reshape(1, HID),
      l1_w11, l1_uw, l1_ub.reshape(1, HID))

    return (x0p, x1_final, x2p)


# speedup vs baseline: 1.0609x; 1.0040x over previous
import jax
import jax.numpy as jnp
from jax.experimental import pallas as pl
from jax.experimental.pallas import tpu as pltpu

N_EDGES = 8192
N_NODES = 8192
N_FACES = 2048
HID = 32
ROW_BLK = 256
NB = N_EDGES // ROW_BLK
I2_BLK = N_FACES // NB * (N_EDGES // N_FACES)
assert I2_BLK * NB == N_EDGES
C_ST0 = 0
C_ST1 = HID


def _elu(x):
    return jnp.where(x > 0, x, jnp.exp(x) - 1.0)


def _dot(a, b):
    return jnp.dot(a, b, preferred_element_type=jnp.float32)


def _body(x0_ref, x1_ref, x2_ref, i1t_ref, i2_ref, a_ref,
          w0_ref, b0_ref, w1_ref, b1_ref, w2_ref, b2_ref,
          w11a_ref, w01ab_ref, w21ab_ref, uwa_ref, uba_ref,
          w11b_ref, uwb_ref, ubb_ref,
          x0p_out, x1_out, x2p_out, y1a_ref, y1b_ref, s_ref,
          x0p16_ref, x2p16_ref):
    i = pl.program_id(0)
    bf16 = jnp.bfloat16

    @pl.when(i == 0)
    def _():
        x0p = _elu(_dot(x0_ref[...], w0_ref[...]) + b0_ref[...])
        x0p_out[...] = x0p
        x0p16_ref[...] = x0p.astype(bf16)
        x2p = _elu(_dot(x2_ref[...], w2_ref[...]) + b2_ref[...])
        x2p_out[...] = x2p
        x2p16_ref[...] = x2p.astype(bf16)

    @pl.when(i < NB)
    def _():
        row = i * ROW_BLK
        p2 = _dot(i2_ref[...].astype(bf16), x2p16_ref[...])
        p0 = _dot(i1t_ref[...].astype(bf16), x0p16_ref[...])
        s_ref[pl.ds(row, ROW_BLK), C_ST0:C_ST0 + 2 * HID] = (
            _elu(_dot(p0, w01ab_ref[...])) + _elu(_dot(p2, w21ab_ref[...])))
        x1p_rows = _elu(
            _dot(x1_ref[pl.ds(row, ROW_BLK), :], w1_ref[...]) + b1_ref[...])
        y1a_ref[pl.ds(row, ROW_BLK), :] = _dot(
            x1p_rows, w11a_ref[...]).astype(bf16)

    @pl.when((i >= NB) & (i < 2 * NB))
    def _():
        row = (i - NB) * ROW_BLK
        x_up = _elu(_dot(a_ref[...].astype(bf16), y1a_ref[...]))
        agg = x_up + s_ref[pl.ds(row, ROW_BLK), C_ST0:C_ST0 + HID]
        x1l0 = _elu(_dot(agg, uwa_ref[...]) + uba_ref[...])
        y1b_ref[pl.ds(row, ROW_BLK), :] = _dot(
            x1l0, w11b_ref[...]).astype(bf16)

    @pl.when(i >= 2 * NB)
    def _():
        row = (i - 2 * NB) * ROW_BLK
        x_up = _elu(_dot(a_ref[...].astype(bf16), y1b_ref[...]))
        agg = x_up + s_ref[pl.ds(row, ROW_BLK), C_ST1:C_ST1 + HID]
        x1_out[...] = _elu(_dot(agg, uwb_ref[...]) + ubb_ref[...])


@jax.jit
def kernel(x_0, x_1, x_2, adjacency_0, incidence_2, incidence_1_t,
           proj0_w, proj0_b, proj1_w, proj1_b, proj2_w, proj2_b,
           l0_w11, l0_w21, l0_w01, l0_uw, l0_ub,
           l1_w11, l1_w21, l1_w01, l1_uw, l1_ub):
    f32 = jnp.float32
    const2 = lambda i: (0, 0)
    C0 = x_0.shape[1]
    C1 = x_1.shape[1]
    C2 = x_2.shape[1]

    w01ab = jnp.concatenate([l0_w01, l1_w01], axis=1)
    w21ab = jnp.concatenate([l0_w21, l1_w21], axis=1)

    def i2_map(i):
        return (jnp.minimum(i, NB - 1), 0)

    def i1t_map(i):
        return (jnp.minimum(i, NB - 1), 0)

    def a_map(i):
        return (jnp.where(i < NB, 0,
                          jnp.where(i < 2 * NB, i - NB, i - 2 * NB)), 0)

    def out_map(i):
        return (jnp.maximum(i - 2 * NB, 0), 0)

    wsq = lambda: pl.BlockSpec((HID, HID), const2)
    bias = lambda: pl.BlockSpec((1, HID), const2)

    x0p, x1_final, x2p = pl.pallas_call(
        _body,
        grid=(3 * NB,),
        in_specs=[
            pl.BlockSpec((N_NODES, C0), const2),
            pl.BlockSpec((N_EDGES, C1), const2),
            pl.BlockSpec((N_FACES, C2), const2),
            pl.BlockSpec((ROW_BLK, N_NODES), i1t_map),
            pl.BlockSpec((I2_BLK, N_FACES), i2_map),
            pl.BlockSpec((ROW_BLK, N_EDGES), a_map),
            pl.BlockSpec((C0, HID), const2),
            bias(),
            pl.BlockSpec((C1, HID), const2),
            bias(),
            pl.BlockSpec((C2, HID), const2),
            bias(),
            wsq(),
            pl.BlockSpec((HID, 2 * HID), const2),
            pl.BlockSpec((HID, 2 * HID), const2),
            wsq(),
            bias(),
            wsq(),
            wsq(),
            bias(),
        ],
        out_specs=(
            pl.BlockSpec((N_NODES, HID), const2),
            pl.BlockSpec((ROW_BLK, HID), out_map),
            pl.BlockSpec((N_FACES, HID), const2),
        ),
        out_shape=(
            jax.ShapeDtypeStruct((N_NODES, HID), f32),
            jax.ShapeDtypeStruct((N_EDGES, HID), f32),
            jax.ShapeDtypeStruct((N_FACES, HID), f32),
        ),
        scratch_shapes=[
            pltpu.VMEM((N_EDGES, HID), jnp.bfloat16),
            pltpu.VMEM((N_EDGES, HID), jnp.bfloat16),
            pltpu.VMEM((N_EDGES, 2 * HID), f32),
            pltpu.VMEM((N_NODES, HID), jnp.bfloat16),
            pltpu.VMEM((N_FACES, HID), jnp.bfloat16),
        ],
        compiler_params=pltpu.CompilerParams(
            dimension_semantics=("arbitrary",),
            vmem_limit_bytes=63 * 1024 * 1024),
    )(x_0, x_1, x_2, incidence_1_t, incidence_2, adjacency_0,
      proj0_w, proj0_b.reshape(1, HID), proj1_w, proj1_b.reshape(1, HID),
      proj2_w, proj2_b.reshape(1, HID),
      l0_w11, w01ab, w21ab, l0_uw, l0_ub.reshape(1, HID),
      l1_w11, l1_uw, l1_ub.reshape(1, HID))

    return (x0p, x1_final, x2p)
